# baseline (speedup 1.0000x reference)
import functools

import jax
import jax.numpy as jnp
from jax import lax
from jax.experimental import pallas as pl
from jax.experimental.pallas import tpu as pltpu
from jax.experimental.pallas import tpu_sc as plsc

Q = 16384
C = 4096
D = 3
QH = Q // 2
QB = 512
NBH = QH // QB
K_LO = 16219

NW = 32
QW = QH // NW
NG = QW // 16


def _nn_idx_kernel(q_ref, kT_ref, idx_ref):
    q = q_ref[...]
    kT = kT_ref[...]

    q2 = jnp.sum(q * q, axis=1)
    b2 = jnp.sum(kT * kT, axis=0)
    abm2 = jax.lax.dot_general(q, -2.0 * kT, (((1,), (0,)), ((), ())),
                               preferred_element_type=jnp.float32)
    minv = jnp.min((q2[:, None] + b2[None, :]) + abm2, axis=1)
    iota = jax.lax.broadcasted_iota(jnp.int32, (QB, C), 1)
    d2 = (q2[:, None] + b2[None, :]) + abm2
    idx = jnp.min(jnp.where(d2 == minv[:, None], iota, C), axis=1)
    idx_ref[...] = idx.reshape(1, 1, QB)


def _sc_gather_kernel(kx_hbm, ky_hbm, kz_hbm, qx_hbm, qy_hbm, qz_hbm,
                      px_hbm, py_hbm, pz_hbm, idx_hbm, m2_hbm, s_hbm,
                      kx_v, ky_v, kz_v, idx_v,
                      qx_v, qy_v, qz_v, px_v, py_v, pz_v, m2_v, s_v):
    wid = lax.axis_index("s") * 2 + lax.axis_index("c")
    base = wid * QW

    pltpu.sync_copy(kx_hbm, kx_v)
    pltpu.sync_copy(ky_hbm, ky_v)
    pltpu.sync_copy(kz_hbm, kz_v)
    pltpu.sync_copy(idx_hbm.at[pl.ds(base, QW)], idx_v)
    pltpu.sync_copy(qx_hbm.at[pl.ds(base, QW)], qx_v)
    pltpu.sync_copy(qy_hbm.at[pl.ds(base, QW)], qy_v)
    pltpu.sync_copy(qz_hbm.at[pl.ds(base, QW)], qz_v)
    pltpu.sync_copy(px_hbm.at[pl.ds(base, QW)], px_v)
    pltpu.sync_copy(py_hbm.at[pl.ds(base, QW)], py_v)
    pltpu.sync_copy(pz_hbm.at[pl.ds(base, QW)], pz_v)

    def body(g, _):
        sl = pl.ds(g * 16, 16)
        iv = idx_v[sl]
        offx = plsc.load_gather(kx_v, [iv]) - qx_v[sl]
        offy = plsc.load_gather(ky_v, [iv]) - qy_v[sl]
        offz = plsc.load_gather(kz_v, [iv]) - qz_v[sl]
        m2_v[sl] = offx * offx + offy * offy + offz * offz

        s = jnp.zeros((16,), jnp.float32)
        for p_ref, off in ((px_v, offx), (py_v, offy), (pz_v, offz)):
            diff = p_ref[sl] - off
            ax = jnp.abs(diff)
            s = s + jnp.where(ax < 1.0, 0.5 * diff * diff, ax - 0.5)
        s_v[sl] = s
        return 0

    lax.fori_loop(0, NG, body, 0)

    pltpu.sync_copy(m2_v, m2_hbm.at[pl.ds(base, QW)])
    pltpu.sync_copy(s_v, s_hbm.at[pl.ds(base, QW)])


def _finalize_kernel(m2a_ref, m2b_ref, sa_ref, sb_ref, out_ref):
    m2_all = jnp.concatenate([m2a_ref[0, :], m2b_ref[0, :]])
    s_all = jnp.concatenate([sa_ref[0, :], sb_ref[0, :]])
    bits = jax.lax.bitcast_convert_type(m2_all, jnp.int32)

    def body(_, carry):
        lo, hi = carry
        mid = lo + (hi - lo) // 2
        cnt = jnp.sum((bits <= mid).astype(jnp.int32))
        take_lo = cnt >= K_LO + 1
        return (jnp.where(take_lo, lo, mid + 1),
                jnp.where(take_lo, mid, hi))

    lo, _ = jax.lax.fori_loop(0, 31, body, (jnp.int32(0), jnp.int32(2**31 - 1)))
    a_lo_bits = lo
    cnt_le = jnp.sum((bits <= a_lo_bits).astype(jnp.int32))
    above_min = jnp.min(jnp.where(bits > a_lo_bits, bits, jnp.int32(2**31 - 1)))
    a_hi_bits = jnp.where(cnt_le >= K_LO + 2, a_lo_bits, above_min)
    a_lo = jnp.sqrt(jax.lax.bitcast_convert_type(a_lo_bits, jnp.float32))
    a_hi = jnp.sqrt(jax.lax.bitcast_convert_type(a_hi_bits, jnp.float32))

    index = jnp.float32(0.99) * jnp.float32(Q - 1)
    thresh = a_lo * (jnp.ceil(index) - index) + a_hi * (index - jnp.floor(index))
    t2 = thresh * thresh

    mask = (m2_all <= t2).astype(jnp.float32)
    cnt = jnp.sum(mask)
    denom = jnp.maximum(cnt * jnp.float32(D), 1.0)
    loss = jnp.sum(s_all * mask) / denom
    out_ref[...] = jnp.reshape(loss, (1, 1))


def _run_tc_argmin(q_half, keysT):
    return pl.pallas_call(
        _nn_idx_kernel,
        grid=(NBH,),
        in_specs=[
            pl.BlockSpec((QB, D), lambda i: (i, 0)),
            pl.BlockSpec((D, C), lambda i: (0, 0)),
        ],
        out_specs=pl.BlockSpec((1, 1, QB), lambda i: (i, 0, 0)),
        out_shape=jax.ShapeDtypeStruct((NBH, 1, QB), jnp.int32),
        compiler_params=pltpu.CompilerParams(
            dimension_semantics=("arbitrary",),
        ),
    )(q_half, keysT).reshape(QH)


def _run_sc_gather(keysT, qT_half, pT_half, nn_idx):
    mesh = plsc.VectorSubcoreMesh(core_axis_name="c", subcore_axis_name="s")
    return pl.kernel(
        _sc_gather_kernel,
        mesh=mesh,
        compiler_params=pltpu.CompilerParams(needs_layout_passes=False),
        out_type=[
            jax.ShapeDtypeStruct((QH,), jnp.float32),
            jax.ShapeDtypeStruct((QH,), jnp.float32),
        ],
        scratch_types=[
            pltpu.VMEM((C,), jnp.float32),
            pltpu.VMEM((C,), jnp.float32),
            pltpu.VMEM((C,), jnp.float32),
            pltpu.VMEM((QW,), jnp.int32),
            pltpu.VMEM((QW,), jnp.float32),
            pltpu.VMEM((QW,), jnp.float32),
            pltpu.VMEM((QW,), jnp.float32),
            pltpu.VMEM((QW,), jnp.float32),
            pltpu.VMEM((QW,), jnp.float32),
            pltpu.VMEM((QW,), jnp.float32),
            pltpu.VMEM((QW,), jnp.float32),
            pltpu.VMEM((QW,), jnp.float32),
        ],
    )(keysT[0], keysT[1], keysT[2], qT_half[0], qT_half[1], qT_half[2],
      pT_half[0], pT_half[1], pT_half[2], nn_idx)


@jax.jit
def kernel(pred_off, queries, keys):
    keysT = keys.T
    qT = queries.T
    pT = pred_off.T

    idx_a = _run_tc_argmin(queries[:QH], keysT)
    m2_a, s_a = _run_sc_gather(keysT, qT[:, :QH], pT[:, :QH], idx_a)
    idx_b = _run_tc_argmin(queries[QH:], keysT)
    m2_b, s_b = _run_sc_gather(keysT, qT[:, QH:], pT[:, QH:], idx_b)

    out = pl.pallas_call(
        _finalize_kernel,
        in_specs=[pl.BlockSpec((1, QH), lambda: (0, 0))] * 4,
        out_specs=pl.BlockSpec((1, 1), lambda: (0, 0)),
        out_shape=jax.ShapeDtypeStruct((1, 1), jnp.float32),
    )(m2_a.reshape(1, QH), m2_b.reshape(1, QH),
      s_a.reshape(1, QH), s_b.reshape(1, QH))
    return out[0, 0]


# speedup vs baseline: 1.4249x; 1.4249x over previous
import functools

import jax
import jax.numpy as jnp
from jax import lax
from jax.experimental import pallas as pl
from jax.experimental.pallas import tpu as pltpu
from jax.experimental.pallas import tpu_sc as plsc

Q = 16384
C = 4096
D = 3
QB = 512
NB = Q // QB
K_LO = 16219

NW = 32
QW = Q // NW
NG = QW // 16


def _nn_idx_kernel(q_ref, kT_ref, idx_ref):
    q = q_ref[...]
    kT = kT_ref[...]

    q2 = jnp.sum(q * q, axis=1)
    b2 = jnp.sum(kT * kT, axis=0)
    abm2 = jax.lax.dot_general(q, -2.0 * kT, (((1,), (0,)), ((), ())),
                               preferred_element_type=jnp.float32)
    d2 = (q2[:, None] + b2[None, :]) + abm2
    idx = jnp.argmin(d2, axis=1).astype(jnp.int32)
    idx_ref[...] = idx.reshape(1, 1, QB)


def _sc_gather_kernel(kx_hbm, ky_hbm, kz_hbm, qx_hbm, qy_hbm, qz_hbm,
                      px_hbm, py_hbm, pz_hbm, idx_hbm, m2_hbm, s_hbm,
                      kx_v, ky_v, kz_v, idx_v,
                      qx_v, qy_v, qz_v, px_v, py_v, pz_v, m2_v, s_v):
    wid = lax.axis_index("s") * 2 + lax.axis_index("c")
    base = wid * QW

    pltpu.sync_copy(kx_hbm, kx_v)
    pltpu.sync_copy(ky_hbm, ky_v)
    pltpu.sync_copy(kz_hbm, kz_v)
    pltpu.sync_copy(idx_hbm.at[pl.ds(base, QW)], idx_v)
    pltpu.sync_copy(qx_hbm.at[pl.ds(base, QW)], qx_v)
    pltpu.sync_copy(qy_hbm.at[pl.ds(base, QW)], qy_v)
    pltpu.sync_copy(qz_hbm.at[pl.ds(base, QW)], qz_v)
    pltpu.sync_copy(px_hbm.at[pl.ds(base, QW)], px_v)
    pltpu.sync_copy(py_hbm.at[pl.ds(base, QW)], py_v)
    pltpu.sync_copy(pz_hbm.at[pl.ds(base, QW)], pz_v)

    def body(g, _):
        sl = pl.ds(g * 16, 16)
        iv = idx_v[sl]
        offx = plsc.load_gather(kx_v, [iv]) - qx_v[sl]
        offy = plsc.load_gather(ky_v, [iv]) - qy_v[sl]
        offz = plsc.load_gather(kz_v, [iv]) - qz_v[sl]
        m2_v[sl] = offx * offx + offy * offy + offz * offz

        s = jnp.zeros((16,), jnp.float32)
        for p_ref, off in ((px_v, offx), (py_v, offy), (pz_v, offz)):
            diff = p_ref[sl] - off
            ax = jnp.abs(diff)
            s = s + jnp.where(ax < 1.0, 0.5 * diff * diff, ax - 0.5)
        s_v[sl] = s
        return 0

    lax.fori_loop(0, NG, body, 0)

    pltpu.sync_copy(m2_v, m2_hbm.at[pl.ds(base, QW)])
    pltpu.sync_copy(s_v, s_hbm.at[pl.ds(base, QW)])


def _finalize_kernel(m2_ref, s_ref, out_ref):
    m2_all = m2_ref[0, :]
    s_all = s_ref[0, :]
    bits = jax.lax.bitcast_convert_type(m2_all, jnp.int32)

    def body(_, carry):
        lo, hi = carry
        mid = lo + (hi - lo) // 2
        cnt = jnp.sum((bits <= mid).astype(jnp.int32))
        take_lo = cnt >= K_LO + 1
        return (jnp.where(take_lo, lo, mid + 1),
                jnp.where(take_lo, mid, hi))

    lo, _ = jax.lax.fori_loop(0, 31, body, (jnp.int32(0), jnp.int32(2**31 - 1)))
    a_lo_bits = lo
    cnt_le = jnp.sum((bits <= a_lo_bits).astype(jnp.int32))
    above_min = jnp.min(jnp.where(bits > a_lo_bits, bits, jnp.int32(2**31 - 1)))
    a_hi_bits = jnp.where(cnt_le >= K_LO + 2, a_lo_bits, above_min)
    a_lo = jnp.sqrt(jax.lax.bitcast_convert_type(a_lo_bits, jnp.float32))
    a_hi = jnp.sqrt(jax.lax.bitcast_convert_type(a_hi_bits, jnp.float32))

    index = jnp.float32(0.99) * jnp.float32(Q - 1)
    thresh = a_lo * (jnp.ceil(index) - index) + a_hi * (index - jnp.floor(index))
    t2 = thresh * thresh

    mask = (m2_all <= t2).astype(jnp.float32)
    cnt = jnp.sum(mask)
    denom = jnp.maximum(cnt * jnp.float32(D), 1.0)
    loss = jnp.sum(s_all * mask) / denom
    out_ref[...] = jnp.reshape(loss, (1, 1))


@jax.jit
def kernel(pred_off, queries, keys):
    keysT = keys.T
    qT = queries.T
    pT = pred_off.T

    nn_idx = pl.pallas_call(
        _nn_idx_kernel,
        grid=(NB,),
        in_specs=[
            pl.BlockSpec((QB, D), lambda i: (i, 0)),
            pl.BlockSpec((D, C), lambda i: (0, 0)),
        ],
        out_specs=pl.BlockSpec((1, 1, QB), lambda i: (i, 0, 0)),
        out_shape=jax.ShapeDtypeStruct((NB, 1, QB), jnp.int32),
        compiler_params=pltpu.CompilerParams(
            dimension_semantics=("arbitrary",),
        ),
    )(queries, keysT)
    nn_idx = nn_idx.reshape(Q)

    mesh = plsc.VectorSubcoreMesh(core_axis_name="c", subcore_axis_name="s")
    m2, s = pl.kernel(
        _sc_gather_kernel,
        mesh=mesh,
        compiler_params=pltpu.CompilerParams(needs_layout_passes=False),
        out_type=[
            jax.ShapeDtypeStruct((Q,), jnp.float32),
            jax.ShapeDtypeStruct((Q,), jnp.float32),
        ],
        scratch_types=[
            pltpu.VMEM((C,), jnp.float32),
            pltpu.VMEM((C,), jnp.float32),
            pltpu.VMEM((C,), jnp.float32),
            pltpu.VMEM((QW,), jnp.int32),
            pltpu.VMEM((QW,), jnp.float32),
            pltpu.VMEM((QW,), jnp.float32),
            pltpu.VMEM((QW,), jnp.float32),
            pltpu.VMEM((QW,), jnp.float32),
            pltpu.VMEM((QW,), jnp.float32),
            pltpu.VMEM((QW,), jnp.float32),
            pltpu.VMEM((QW,), jnp.float32),
            pltpu.VMEM((QW,), jnp.float32),
        ],
    )(keysT[0], keysT[1], keysT[2], qT[0], qT[1], qT[2],
      pT[0], pT[1], pT[2], nn_idx)

    out = pl.pallas_call(
        _finalize_kernel,
        in_specs=[
            pl.BlockSpec((1, Q), lambda: (0, 0)),
            pl.BlockSpec((1, Q), lambda: (0, 0)),
        ],
        out_specs=pl.BlockSpec((1, 1), lambda: (0, 0)),
        out_shape=jax.ShapeDtypeStruct((1, 1), jnp.float32),
    )(m2.reshape(1, Q), s.reshape(1, Q))
    return out[0, 0]


# speedup vs baseline: 1.4335x; 1.0061x over previous
import functools

import jax
import jax.numpy as jnp
from jax import lax
from jax.experimental import pallas as pl
from jax.experimental.pallas import tpu as pltpu
from jax.experimental.pallas import tpu_sc as plsc

Q = 16384
C = 4096
D = 3
QB = 1024
NB = Q // QB
K_LO = 16219

NW = 32
QW = Q // NW
NG = QW // 16


def _nn_idx_kernel(q_ref, kT_ref, idx_ref):
    q = q_ref[...]
    kT = kT_ref[...]

    q2 = jnp.sum(q * q, axis=1)
    b2 = jnp.sum(kT * kT, axis=0)
    abm2 = jax.lax.dot_general(q, -2.0 * kT, (((1,), (0,)), ((), ())),
                               preferred_element_type=jnp.float32)
    d2 = (q2[:, None] + b2[None, :]) + abm2
    idx = jnp.argmin(d2, axis=1).astype(jnp.int32)
    idx_ref[...] = idx.reshape(1, 1, QB)


def _sc_gather_kernel(kx_hbm, ky_hbm, kz_hbm, qx_hbm, qy_hbm, qz_hbm,
                      px_hbm, py_hbm, pz_hbm, idx_hbm, m2_hbm, s_hbm,
                      kx_v, ky_v, kz_v, idx_v,
                      qx_v, qy_v, qz_v, px_v, py_v, pz_v, m2_v, s_v):
    wid = lax.axis_index("s") * 2 + lax.axis_index("c")
    base = wid * QW

    pltpu.sync_copy(kx_hbm, kx_v)
    pltpu.sync_copy(ky_hbm, ky_v)
    pltpu.sync_copy(kz_hbm, kz_v)
    pltpu.sync_copy(idx_hbm.at[pl.ds(base, QW)], idx_v)
    pltpu.sync_copy(qx_hbm.at[pl.ds(base, QW)], qx_v)
    pltpu.sync_copy(qy_hbm.at[pl.ds(base, QW)], qy_v)
    pltpu.sync_copy(qz_hbm.at[pl.ds(base, QW)], qz_v)
    pltpu.sync_copy(px_hbm.at[pl.ds(base, QW)], px_v)
    pltpu.sync_copy(py_hbm.at[pl.ds(base, QW)], py_v)
    pltpu.sync_copy(pz_hbm.at[pl.ds(base, QW)], pz_v)

    def body(g, _):
        sl = pl.ds(g * 16, 16)
        iv = idx_v[sl]
        offx = plsc.load_gather(kx_v, [iv]) - qx_v[sl]
        offy = plsc.load_gather(ky_v, [iv]) - qy_v[sl]
        offz = plsc.load_gather(kz_v, [iv]) - qz_v[sl]
        m2_v[sl] = offx * offx + offy * offy + offz * offz

        s = jnp.zeros((16,), jnp.float32)
        for p_ref, off in ((px_v, offx), (py_v, offy), (pz_v, offz)):
            diff = p_ref[sl] - off
            ax = jnp.abs(diff)
            s = s + jnp.where(ax < 1.0, 0.5 * diff * diff, ax - 0.5)
        s_v[sl] = s
        return 0

    lax.fori_loop(0, NG, body, 0)

    pltpu.sync_copy(m2_v, m2_hbm.at[pl.ds(base, QW)])
    pltpu.sync_copy(s_v, s_hbm.at[pl.ds(base, QW)])


def _finalize_kernel(m2_ref, s_ref, out_ref):
    m2_all = m2_ref[0, :]
    s_all = s_ref[0, :]
    bits = jax.lax.bitcast_convert_type(m2_all, jnp.int32)

    def body(_, carry):
        lo, hi = carry
        mid = lo + (hi - lo) // 2
        cnt = jnp.sum((bits <= mid).astype(jnp.int32))
        take_lo = cnt >= K_LO + 1
        return (jnp.where(take_lo, lo, mid + 1),
                jnp.where(take_lo, mid, hi))

    lo, _ = jax.lax.fori_loop(0, 31, body, (jnp.int32(0), jnp.int32(2**31 - 1)))
    a_lo_bits = lo
    cnt_le = jnp.sum((bits <= a_lo_bits).astype(jnp.int32))
    above_min = jnp.min(jnp.where(bits > a_lo_bits, bits, jnp.int32(2**31 - 1)))
    a_hi_bits = jnp.where(cnt_le >= K_LO + 2, a_lo_bits, above_min)
    a_lo = jnp.sqrt(jax.lax.bitcast_convert_type(a_lo_bits, jnp.float32))
    a_hi = jnp.sqrt(jax.lax.bitcast_convert_type(a_hi_bits, jnp.float32))

    index = jnp.float32(0.99) * jnp.float32(Q - 1)
    thresh = a_lo * (jnp.ceil(index) - index) + a_hi * (index - jnp.floor(index))
    t2 = thresh * thresh

    mask = (m2_all <= t2).astype(jnp.float32)
    cnt = jnp.sum(mask)
    denom = jnp.maximum(cnt * jnp.float32(D), 1.0)
    loss = jnp.sum(s_all * mask) / denom
    out_ref[...] = jnp.reshape(loss, (1, 1))


@jax.jit
def kernel(pred_off, queries, keys):
    keysT = keys.T
    qT = queries.T
    pT = pred_off.T

    nn_idx = pl.pallas_call(
        _nn_idx_kernel,
        grid=(NB,),
        in_specs=[
            pl.BlockSpec((QB, D), lambda i: (i, 0)),
            pl.BlockSpec((D, C), lambda i: (0, 0)),
        ],
        out_specs=pl.BlockSpec((1, 1, QB), lambda i: (i, 0, 0)),
        out_shape=jax.ShapeDtypeStruct((NB, 1, QB), jnp.int32),
        compiler_params=pltpu.CompilerParams(
            dimension_semantics=("arbitrary",),
        ),
    )(queries, keysT)
    nn_idx = nn_idx.reshape(Q)

    mesh = plsc.VectorSubcoreMesh(core_axis_name="c", subcore_axis_name="s")
    m2, s = pl.kernel(
        _sc_gather_kernel,
        mesh=mesh,
        compiler_params=pltpu.CompilerParams(needs_layout_passes=False),
        out_type=[
            jax.ShapeDtypeStruct((Q,), jnp.float32),
            jax.ShapeDtypeStruct((Q,), jnp.float32),
        ],
        scratch_types=[
            pltpu.VMEM((C,), jnp.float32),
            pltpu.VMEM((C,), jnp.float32),
            pltpu.VMEM((C,), jnp.float32),
            pltpu.VMEM((QW,), jnp.int32),
            pltpu.VMEM((QW,), jnp.float32),
            pltpu.VMEM((QW,), jnp.float32),
            pltpu.VMEM((QW,), jnp.float32),
            pltpu.VMEM((QW,), jnp.float32),
            pltpu.VMEM((QW,), jnp.float32),
            pltpu.VMEM((QW,), jnp.float32),
            pltpu.VMEM((QW,), jnp.float32),
            pltpu.VMEM((QW,), jnp.float32),
        ],
    )(keysT[0], keysT[1], keysT[2], qT[0], qT[1], qT[2],
      pT[0], pT[1], pT[2], nn_idx)

    out = pl.pallas_call(
        _finalize_kernel,
        in_specs=[
            pl.BlockSpec((1, Q), lambda: (0, 0)),
            pl.BlockSpec((1, Q), lambda: (0, 0)),
        ],
        out_specs=pl.BlockSpec((1, 1), lambda: (0, 0)),
        out_shape=jax.ShapeDtypeStruct((1, 1), jnp.float32),
    )(m2.reshape(1, Q), s.reshape(1, Q))
    return out[0, 0]
